# Initial kernel scaffold; baseline (speedup 1.0000x reference)
import jax
import jax.numpy as jnp
from jax.experimental import pallas as pl


def kernel(input_batch_inputs_, weight, mat1, mat2):
    raise NotImplementedError("write your pallas kernel here")


# speedup vs baseline: 1.2678x; 1.2678x over previous
import functools

import jax
import jax.numpy as jnp
from jax import lax
from jax.experimental import pallas as pl
from jax.experimental.pallas import tpu as pltpu
from jax.experimental.pallas import tpu_sc as plsc

_NUM_CORES = 2
_NUM_SUBCORES = 16
_NW = _NUM_CORES * _NUM_SUBCORES


@functools.lru_cache(maxsize=None)
def _make_gather(num_rows: int, dim: int, batch: int):
    assert batch % _NW == 0
    bpw = batch // _NW
    ch = 128
    while bpw % ch != 0:
        ch //= 2
    nch = bpw // ch
    assert nch % 2 == 0 or nch == 1

    mesh = plsc.VectorSubcoreMesh(core_axis_name="c", subcore_axis_name="s")

    @functools.partial(
        pl.kernel,
        out_type=jax.ShapeDtypeStruct((batch, dim), jnp.float32),
        mesh=mesh,
        scratch_types=[
            pltpu.VMEM((bpw,), jnp.int32),
            pltpu.VMEM((ch, dim), jnp.float32),
            pltpu.VMEM((ch, dim), jnp.float32),
            pltpu.SemaphoreType.DMA,
            pltpu.SemaphoreType.DMA,
        ],
    )
    def gather(idx_hbm, tbl_hbm, out_hbm, idx_v, buf0, buf1, sem0, sem1):
        wid = lax.axis_index("s") * _NUM_CORES + lax.axis_index("c")
        base = wid * bpw
        pltpu.sync_copy(idx_hbm.at[pl.ds(base, bpw)], idx_v)

        def body(h, carry):
            g = pl.multiple_of(2 * h, 2)
            o0 = pl.multiple_of(g * ch, 8)
            o1 = pl.multiple_of((g + 1) * ch, 8)
            d0 = pltpu.async_copy(tbl_hbm.at[idx_v.at[pl.ds(o0, ch)]], buf0, sem0)
            d1 = pltpu.async_copy(tbl_hbm.at[idx_v.at[pl.ds(o1, ch)]], buf1, sem1)
            d0.wait()
            pltpu.sync_copy(buf0, out_hbm.at[pl.ds(base + o0, ch)])
            d1.wait()
            pltpu.sync_copy(buf1, out_hbm.at[pl.ds(base + o1, ch)])
            return carry

        lax.fori_loop(0, nch // 2, body, 0, unroll=False)

    return gather


def _mm_body(a_ref, b_ref, o_ref):
    o_ref[...] = jnp.dot(a_ref[...], b_ref[...],
                         preferred_element_type=jnp.float32)


@functools.lru_cache(maxsize=None)
def _make_mm(m: int, k: int, n: int):
    return pl.pallas_call(
        _mm_body,
        out_shape=jax.ShapeDtypeStruct((m, n), jnp.float32),
    )


@jax.jit
def kernel(input_batch_inputs_, weight, mat1, mat2):
    bsz, fields = input_batch_inputs_.shape
    num_rows, dim = weight.shape
    idx = input_batch_inputs_.reshape(-1)
    emb_flat = _make_gather(num_rows, dim, bsz * fields)(idx, weight)
    emb = emb_flat.reshape(bsz, fields, dim)
    mm = _make_mm(mat1.shape[0], mat1.shape[1], mat2.shape[1])(mat1, mat2)
    return emb, mm


# speedup vs baseline: 1.2829x; 1.0119x over previous
import functools

import jax
import jax.numpy as jnp
from jax import lax
from jax.experimental import pallas as pl
from jax.experimental.pallas import tpu as pltpu
from jax.experimental.pallas import tpu_sc as plsc

_NUM_CORES = 2
_NUM_SUBCORES = 16
_NW = _NUM_CORES * _NUM_SUBCORES


def _pick_chunk(bpw: int, dim: int) -> int:
    budget = 480 * 1024 - bpw * 4
    ch = 8
    for cand in range(8, bpw + 1, 8):
        if bpw % cand == 0 and 2 * cand * dim * 4 <= budget:
            ch = cand
    return ch


@functools.lru_cache(maxsize=None)
def _make_gather(num_rows: int, dim: int, batch: int):
    assert batch % _NW == 0
    bpw = batch // _NW
    ch = _pick_chunk(bpw, dim)
    nch = bpw // ch

    mesh = plsc.VectorSubcoreMesh(core_axis_name="c", subcore_axis_name="s")

    @functools.partial(
        pl.kernel,
        out_type=jax.ShapeDtypeStruct((batch, dim), jnp.float32),
        mesh=mesh,
        scratch_types=[
            pltpu.VMEM((bpw,), jnp.int32),
            pltpu.VMEM((ch, dim), jnp.float32),
            pltpu.VMEM((ch, dim), jnp.float32),
            pltpu.SemaphoreType.DMA,
            pltpu.SemaphoreType.DMA,
            pltpu.SemaphoreType.DMA,
            pltpu.SemaphoreType.DMA,
        ],
    )
    def gather(idx_hbm, tbl_hbm, out_hbm, idx_v, buf0, buf1,
               gs0, gs1, ws0, ws1):
        wid = lax.axis_index("s") * _NUM_CORES + lax.axis_index("c")
        base = wid * bpw
        pltpu.sync_copy(idx_hbm.at[pl.ds(base, bpw)], idx_v)

        bufs, gsems, wsems = (buf0, buf1), (gs0, gs1), (ws0, ws1)
        gd = [None] * nch
        wd = [None] * nch
        for g in range(nch):
            b = g & 1
            if g >= 2:
                wd[g - 2].wait()
            gd[g] = pltpu.async_copy(
                tbl_hbm.at[idx_v.at[pl.ds(g * ch, ch)]], bufs[b], gsems[b])
            if g >= 1:
                p = g - 1
                gd[p].wait()
                wd[p] = pltpu.async_copy(
                    bufs[p & 1], out_hbm.at[pl.ds(base + p * ch, ch)],
                    wsems[p & 1])
        p = nch - 1
        gd[p].wait()
        wd[p] = pltpu.async_copy(
            bufs[p & 1], out_hbm.at[pl.ds(base + p * ch, ch)], wsems[p & 1])
        if nch >= 2:
            wd[nch - 2].wait()
        wd[nch - 1].wait()

    return gather


def _mm_body(a_ref, b_ref, o_ref):
    o_ref[...] = jnp.dot(a_ref[...], b_ref[...],
                         preferred_element_type=jnp.float32)


@functools.lru_cache(maxsize=None)
def _make_mm(m: int, k: int, n: int):
    return pl.pallas_call(
        _mm_body,
        out_shape=jax.ShapeDtypeStruct((m, n), jnp.float32),
    )


@jax.jit
def kernel(input_batch_inputs_, weight, mat1, mat2):
    bsz, fields = input_batch_inputs_.shape
    num_rows, dim = weight.shape
    idx = input_batch_inputs_.reshape(-1)
    emb_flat = _make_gather(num_rows, dim, bsz * fields)(idx, weight)
    emb = emb_flat.reshape(bsz, fields, dim)
    mm = _make_mm(mat1.shape[0], mat1.shape[1], mat2.shape[1])(mat1, mat2)
    return emb, mm


# speedup vs baseline: 1.2841x; 1.0009x over previous
import functools

import jax
import jax.numpy as jnp
from jax import lax
from jax.experimental import pallas as pl
from jax.experimental.pallas import tpu as pltpu
from jax.experimental.pallas import tpu_sc as plsc

_NUM_CORES = 2
_NUM_SUBCORES = 16
_NW = _NUM_CORES * _NUM_SUBCORES


def _pick_chunk(bpw: int, dim: int) -> int:
    budget = 480 * 1024 - bpw * 4
    ch = 8
    for cand in range(8, bpw + 1, 8):
        if bpw % cand == 0 and 2 * cand * dim * 4 <= budget:
            ch = cand
    return ch


@functools.lru_cache(maxsize=None)
def _make_gather(num_rows: int, dim: int, batch: int):
    assert batch % _NW == 0
    bpw = batch // _NW
    ch = _pick_chunk(bpw, dim)
    nch = bpw // ch

    mesh = plsc.VectorSubcoreMesh(core_axis_name="c", subcore_axis_name="s")

    @functools.partial(
        pl.kernel,
        out_type=jax.ShapeDtypeStruct((batch, dim), jnp.float32),
        mesh=mesh,
        compiler_params=pltpu.CompilerParams(use_tc_tiling_on_sc=True),
        scratch_types=[
            pltpu.VMEM((bpw,), jnp.int32),
            pltpu.VMEM((ch, dim), jnp.float32),
            pltpu.VMEM((ch, dim), jnp.float32),
            pltpu.SemaphoreType.DMA,
            pltpu.SemaphoreType.DMA,
            pltpu.SemaphoreType.DMA,
            pltpu.SemaphoreType.DMA,
        ],
    )
    def gather(idx_hbm, tbl_hbm, out_hbm, idx_v, buf0, buf1,
               gs0, gs1, ws0, ws1):
        wid = lax.axis_index("s") * _NUM_CORES + lax.axis_index("c")
        base = wid * bpw
        pltpu.sync_copy(idx_hbm.at[pl.ds(base, bpw)], idx_v)

        bufs, gsems, wsems = (buf0, buf1), (gs0, gs1), (ws0, ws1)
        gd = [None] * nch
        wd = [None] * nch
        for g in range(nch):
            b = g & 1
            if g >= 2:
                wd[g - 2].wait()
            gd[g] = pltpu.async_copy(
                tbl_hbm.at[idx_v.at[pl.ds(g * ch, ch)]], bufs[b], gsems[b])
            if g >= 1:
                p = g - 1
                gd[p].wait()
                wd[p] = pltpu.async_copy(
                    bufs[p & 1], out_hbm.at[pl.ds(base + p * ch, ch)],
                    wsems[p & 1])
        p = nch - 1
        gd[p].wait()
        wd[p] = pltpu.async_copy(
            bufs[p & 1], out_hbm.at[pl.ds(base + p * ch, ch)], wsems[p & 1])
        if nch >= 2:
            wd[nch - 2].wait()
        wd[nch - 1].wait()

    return gather


def _mm_body(a_ref, b_ref, o_ref):
    o_ref[...] = jnp.dot(a_ref[...], b_ref[...],
                         preferred_element_type=jnp.float32)


@functools.lru_cache(maxsize=None)
def _make_mm(m: int, k: int, n: int):
    return pl.pallas_call(
        _mm_body,
        out_shape=jax.ShapeDtypeStruct((m, n), jnp.float32),
    )


@jax.jit
def kernel(input_batch_inputs_, weight, mat1, mat2):
    bsz, fields = input_batch_inputs_.shape
    num_rows, dim = weight.shape
    idx = input_batch_inputs_.reshape(-1)
    emb_flat = _make_gather(num_rows, dim, bsz * fields)(idx, weight)
    emb = emb_flat.reshape(bsz, fields, dim)
    mm = _make_mm(mat1.shape[0], mat1.shape[1], mat2.shape[1])(mat1, mat2)
    return emb, mm


# speedup vs baseline: 1.7959x; 1.3985x over previous
import functools

import jax
import jax.numpy as jnp
from jax import lax
from jax.experimental import pallas as pl
from jax.experimental.pallas import tpu as pltpu
from jax.experimental.pallas import tpu_sc as plsc

_NUM_CORES = 2
_NUM_SUBCORES = 16
_NW = _NUM_CORES * _NUM_SUBCORES

_NBUF = 8


def _round_up(x: int, m: int) -> int:
    return (x + m - 1) // m * m


@functools.lru_cache(maxsize=None)
def _make_gather(num_rows: int, dim: int, bsz: int, fields: int):
    assert bsz % _NW == 0
    fpad = _round_up(fields, 8)
    ppw = bsz // _NW
    ipw = ppw * fpad

    mesh = plsc.VectorSubcoreMesh(core_axis_name="c", subcore_axis_name="s")

    @functools.partial(
        pl.kernel,
        out_type=jax.ShapeDtypeStruct((bsz, fields, dim), jnp.float32),
        mesh=mesh,
        compiler_params=pltpu.CompilerParams(use_tc_tiling_on_sc=True),
        scratch_types=[
            pltpu.VMEM((ipw,), jnp.int32),
            [pltpu.VMEM((fields, dim), jnp.float32) for _ in range(_NBUF)],
            [pltpu.SemaphoreType.DMA for _ in range(_NBUF)],
            [pltpu.SemaphoreType.DMA for _ in range(_NBUF)],
        ],
    )
    def gather(idx_hbm, tbl_hbm, out_hbm, idx_v, bufs, gsems, wsems):
        wid = lax.axis_index("s") * _NUM_CORES + lax.axis_index("c")
        pbase = wid * ppw
        pltpu.sync_copy(idx_hbm.at[pl.ds(wid * ipw, ipw)], idx_v)

        gd = [None] * ppw
        wd = [None] * ppw
        for p in range(ppw):
            b = p % _NBUF
            if p >= _NBUF:
                wd[p - _NBUF].wait()
            gd[p] = pltpu.async_copy(
                tbl_hbm.at[idx_v.at[pl.ds(p * fpad, fields)]],
                bufs[b], gsems[b])
            if p >= 1:
                gd[p - 1].wait()
                wd[p - 1] = pltpu.async_copy(
                    bufs[(p - 1) % _NBUF], out_hbm.at[pbase + p - 1],
                    wsems[(p - 1) % _NBUF])
        gd[ppw - 1].wait()
        wd[ppw - 1] = pltpu.async_copy(
            bufs[(ppw - 1) % _NBUF], out_hbm.at[pbase + ppw - 1],
            wsems[(ppw - 1) % _NBUF])
        for p in range(max(0, ppw - _NBUF + 1), ppw):
            wd[p].wait()

    return gather


def _mm_body(a_ref, b_ref, o_ref):
    o_ref[...] = jnp.dot(a_ref[...], b_ref[...],
                         preferred_element_type=jnp.float32)


@functools.lru_cache(maxsize=None)
def _make_mm(m: int, k: int, n: int):
    return pl.pallas_call(
        _mm_body,
        out_shape=jax.ShapeDtypeStruct((m, n), jnp.float32),
    )


@jax.jit
def kernel(input_batch_inputs_, weight, mat1, mat2):
    bsz, fields = input_batch_inputs_.shape
    num_rows, dim = weight.shape
    fpad = _round_up(fields, 8)
    idx_pad = jnp.pad(input_batch_inputs_,
                      ((0, 0), (0, fpad - fields))).reshape(-1)
    emb = _make_gather(num_rows, dim, bsz, fields)(idx_pad, weight)
    mm = _make_mm(mat1.shape[0], mat1.shape[1], mat2.shape[1])(mat1, mat2)
    return emb, mm
